# Initial kernel scaffold; baseline (speedup 1.0000x reference)
import jax
import jax.numpy as jnp
from jax.experimental import pallas as pl


def kernel(emb, w, b, token_ids, h0, c0):
    raise NotImplementedError("write your pallas kernel here")


# speedup vs baseline: 5.2991x; 5.2991x over previous
import functools

import jax
import jax.numpy as jnp
from jax.experimental import pallas as pl
from jax.experimental.pallas import tpu as pltpu

_H = 256
_G4 = 4 * _H


def _proj_kernel(x_ref, wih_ref, b_ref, g_ref):
    g_ref[...] = (
        jnp.dot(x_ref[...], wih_ref[...], preferred_element_type=jnp.float32)
        + b_ref[...]
    )


def _recur_kernel(g_ref, whh_ref, h0_ref, c0_ref, out_ref, c_out_ref,
                  h_scr, c_scr, *, block_t):
    blk = pl.program_id(0)

    @pl.when(blk == 0)
    def _():
        h_scr[...] = h0_ref[...]
        c_scr[...] = c0_ref[...]

    h = h_scr[...]
    c = c_scr[...]
    for j in range(block_t):
        gates = (
            jnp.dot(h.astype(jnp.bfloat16), whh_ref[...],
                    preferred_element_type=jnp.float32)
            + g_ref[j:j + 1, :]
        )
        ifo = jax.nn.sigmoid(gates[:, :3 * _H])
        g_g = jnp.tanh(gates[:, 3 * _H:])
        c = ifo[:, _H:2 * _H] * c + ifo[:, :_H] * g_g
        h = ifo[:, 2 * _H:3 * _H] * jnp.tanh(c)
        out_ref[j:j + 1, :] = h
    h_scr[...] = h
    c_scr[...] = c

    @pl.when(blk == pl.num_programs(0) - 1)
    def _():
        c_out_ref[...] = c_scr[...]


def _pick_block(n, candidates):
    for c in candidates:
        if n % c == 0:
            return c
    return 1


def kernel(emb, w, b, token_ids, h0, c0):
    S = token_ids.shape[0]
    H = _H

    x = emb[token_ids.astype(jnp.int32)]

    perm = jnp.concatenate([w[:, :2 * H], w[:, 3 * H:], w[:, 2 * H:3 * H]],
                           axis=1)
    b_r = jnp.concatenate([b[:, :2 * H], b[:, 3 * H:], b[:, 2 * H:3 * H]],
                          axis=1)
    wih = perm[:H].astype(jnp.bfloat16)
    whh = perm[H:2 * H].astype(jnp.bfloat16)
    x_bf = x.astype(jnp.bfloat16)

    bt1 = _pick_block(S, (256, 128, 64, 32, 16, 8))
    s_pad = S
    if S % bt1:
        s_pad = (S // bt1 + 1) * bt1
        x_bf = jnp.zeros((s_pad, H), x_bf.dtype).at[:S].set(x_bf)
    gmat = pl.pallas_call(
        _proj_kernel,
        grid=(s_pad // bt1,),
        in_specs=[
            pl.BlockSpec((bt1, H), lambda i: (i, 0)),
            pl.BlockSpec((H, _G4), lambda i: (0, 0)),
            pl.BlockSpec((1, _G4), lambda i: (0, 0)),
        ],
        out_specs=pl.BlockSpec((bt1, _G4), lambda i: (i, 0)),
        out_shape=jax.ShapeDtypeStruct((s_pad, _G4), jnp.float32),
        compiler_params=pltpu.CompilerParams(
            dimension_semantics=("parallel",)),
    )(x_bf, wih, b_r)
    if s_pad != S:
        gmat = gmat[:S]

    bt2 = _pick_block(S, (32, 16, 8, 4, 2))
    h0_2d = h0.reshape(1, H)
    c0_2d = c0.reshape(1, H)
    outputs, c_final = pl.pallas_call(
        functools.partial(_recur_kernel, block_t=bt2),
        grid=(S // bt2,),
        in_specs=[
            pl.BlockSpec((bt2, _G4), lambda i: (i, 0)),
            pl.BlockSpec((H, _G4), lambda i: (0, 0)),
            pl.BlockSpec((1, H), lambda i: (0, 0)),
            pl.BlockSpec((1, H), lambda i: (0, 0)),
        ],
        out_specs=[
            pl.BlockSpec((bt2, H), lambda i: (i, 0)),
            pl.BlockSpec((1, H), lambda i: (0, 0)),
        ],
        out_shape=(jax.ShapeDtypeStruct((S, H), jnp.float32),
                   jax.ShapeDtypeStruct((1, H), jnp.float32)),
        scratch_shapes=[
            pltpu.VMEM((1, H), jnp.float32),
            pltpu.VMEM((1, H), jnp.float32),
        ],
        compiler_params=pltpu.CompilerParams(
            dimension_semantics=("arbitrary",)),
    )(gmat, whh, h0_2d, c0_2d)

    outputs = outputs.reshape(S, 1, H)
    h_final = outputs[-1].reshape(1, 1, H)
    c_final = c_final.reshape(1, 1, H)
    return outputs, (h_final, c_final)


# speedup vs baseline: 5.3522x; 1.0100x over previous
import functools

import jax
import jax.numpy as jnp
from jax.experimental import pallas as pl
from jax.experimental.pallas import tpu as pltpu

_H = 256
_G4 = 4 * _H


def _prep_kernel(w_ref, b_ref, wih_ref, whh_ref, br_ref):
    w = w_ref[...]
    wp = jnp.concatenate(
        [w[:, :2 * _H], w[:, 3 * _H:], w[:, 2 * _H:3 * _H]], axis=1)
    wp = wp.astype(jnp.bfloat16)
    wih_ref[...] = wp[:_H]
    whh_ref[...] = wp[_H:]
    b = b_ref[...]
    br_ref[...] = jnp.concatenate(
        [b[:, :2 * _H], b[:, 3 * _H:], b[:, 2 * _H:3 * _H]], axis=1)


def _proj_kernel(x_ref, wih_ref, b_ref, g_ref):
    g_ref[...] = (
        jnp.dot(x_ref[...].astype(jnp.bfloat16), wih_ref[...],
                preferred_element_type=jnp.float32)
        + b_ref[...]
    )


def _recur_kernel(g_ref, whh_ref, h0_ref, c0_ref, out_ref, c_out_ref,
                  h_scr, c_scr, *, block_t):
    blk = pl.program_id(0)

    @pl.when(blk == 0)
    def _():
        h_scr[...] = h0_ref[...]
        c_scr[...] = c0_ref[...]

    h = h_scr[...]
    c = c_scr[...]
    for j in range(block_t):
        gates = (
            jnp.dot(h.astype(jnp.bfloat16), whh_ref[...],
                    preferred_element_type=jnp.float32)
            + g_ref[j:j + 1, :]
        )
        ifo = jax.nn.sigmoid(gates[:, :3 * _H])
        g_g = jnp.tanh(gates[:, 3 * _H:])
        c = ifo[:, _H:2 * _H] * c + ifo[:, :_H] * g_g
        h = ifo[:, 2 * _H:3 * _H] * jnp.tanh(c)
        out_ref[j:j + 1, :] = h
    h_scr[...] = h
    c_scr[...] = c

    @pl.when(blk == pl.num_programs(0) - 1)
    def _():
        c_out_ref[...] = c_scr[...]


def _pick_block(n, candidates):
    for c in candidates:
        if n % c == 0:
            return c
    return 1


def kernel(emb, w, b, token_ids, h0, c0):
    S = token_ids.shape[0]
    H = _H

    x = emb[token_ids.astype(jnp.int32)]

    wih, whh, b_r = pl.pallas_call(
        _prep_kernel,
        grid=(1,),
        in_specs=[
            pl.BlockSpec((2 * H, _G4), lambda i: (0, 0)),
            pl.BlockSpec((1, _G4), lambda i: (0, 0)),
        ],
        out_specs=[
            pl.BlockSpec((H, _G4), lambda i: (0, 0)),
            pl.BlockSpec((H, _G4), lambda i: (0, 0)),
            pl.BlockSpec((1, _G4), lambda i: (0, 0)),
        ],
        out_shape=(jax.ShapeDtypeStruct((H, _G4), jnp.bfloat16),
                   jax.ShapeDtypeStruct((H, _G4), jnp.bfloat16),
                   jax.ShapeDtypeStruct((1, _G4), jnp.float32)),
    )(w, b)

    bt1 = _pick_block(S, (256, 128, 64, 32, 16, 8))
    s_pad = S
    if S % bt1:
        s_pad = (S // bt1 + 1) * bt1
        x = jnp.zeros((s_pad, H), x.dtype).at[:S].set(x)
    gmat = pl.pallas_call(
        _proj_kernel,
        grid=(s_pad // bt1,),
        in_specs=[
            pl.BlockSpec((bt1, H), lambda i: (i, 0)),
            pl.BlockSpec((H, _G4), lambda i: (0, 0)),
            pl.BlockSpec((1, _G4), lambda i: (0, 0)),
        ],
        out_specs=pl.BlockSpec((bt1, _G4), lambda i: (i, 0)),
        out_shape=jax.ShapeDtypeStruct((s_pad, _G4), jnp.float32),
        compiler_params=pltpu.CompilerParams(
            dimension_semantics=("parallel",)),
    )(x, wih, b_r)
    if s_pad != S:
        gmat = gmat[:S]

    bt2 = _pick_block(S, (64, 32, 16, 8, 4, 2))
    h0_2d = h0.reshape(1, H)
    c0_2d = c0.reshape(1, H)
    outputs, c_final = pl.pallas_call(
        functools.partial(_recur_kernel, block_t=bt2),
        grid=(S // bt2,),
        in_specs=[
            pl.BlockSpec((bt2, _G4), lambda i: (i, 0)),
            pl.BlockSpec((H, _G4), lambda i: (0, 0)),
            pl.BlockSpec((1, H), lambda i: (0, 0)),
            pl.BlockSpec((1, H), lambda i: (0, 0)),
        ],
        out_specs=[
            pl.BlockSpec((bt2, H), lambda i: (i, 0)),
            pl.BlockSpec((1, H), lambda i: (0, 0)),
        ],
        out_shape=(jax.ShapeDtypeStruct((S, H), jnp.float32),
                   jax.ShapeDtypeStruct((1, H), jnp.float32)),
        scratch_shapes=[
            pltpu.VMEM((1, H), jnp.float32),
            pltpu.VMEM((1, H), jnp.float32),
        ],
        compiler_params=pltpu.CompilerParams(
            dimension_semantics=("arbitrary",)),
    )(gmat, whh, h0_2d, c0_2d)

    outputs = outputs.reshape(S, 1, H)
    h_final = outputs[-1].reshape(1, 1, H)
    c_final = c_final.reshape(1, 1, H)
    return outputs, (h_final, c_final)


# speedup vs baseline: 6.1985x; 1.1581x over previous
import functools

import jax
import jax.numpy as jnp
from jax.experimental import pallas as pl
from jax.experimental.pallas import tpu as pltpu

_H = 256
_G4 = 4 * _H


def _prep_kernel(w_ref, wih_ref):
    wih_ref[...] = w_ref[...].astype(jnp.bfloat16)


def _proj_kernel(x_ref, wih_ref, b_ref, g_ref):
    g_ref[...] = (
        jnp.dot(x_ref[...].astype(jnp.bfloat16), wih_ref[...],
                preferred_element_type=jnp.float32)
        + b_ref[...]
    )


def _push_tiles(whh_ref):
    pltpu.matmul_push_rhs(whh_ref[:, 0 * _H:1 * _H],
                          staging_register=0, mxu_index=0)
    pltpu.matmul_push_rhs(whh_ref[:, 2 * _H:3 * _H],
                          staging_register=0, mxu_index=1)
    pltpu.matmul_push_rhs(whh_ref[:, 1 * _H:2 * _H],
                          staging_register=1, mxu_index=0)
    pltpu.matmul_push_rhs(whh_ref[:, 3 * _H:4 * _H],
                          staging_register=1, mxu_index=1)


def _recur_kernel(g_ref, whh_ref, h0_ref, c0_ref, out_ref, c_out_ref,
                  h_scr, c_scr, *, block_t):
    blk = pl.program_id(0)

    @pl.when(blk == 0)
    def _():
        h_scr[...] = jnp.broadcast_to(h0_ref[...], (8, _H))
        c_scr[...] = jnp.broadcast_to(c0_ref[...], (8, _H))

    _push_tiles(whh_ref)

    h = h_scr[...]
    c = c_scr[...]
    for j in range(block_t):
        pltpu.matmul_acc_lhs(0, h, mxu_index=0, load_staged_rhs=0)
        pltpu.matmul_acc_lhs(0, h, mxu_index=1, load_staged_rhs=0)
        pltpu.matmul_acc_lhs(8, h, mxu_index=0, load_staged_rhs=1)
        pltpu.matmul_acc_lhs(8, h, mxu_index=1, load_staged_rhs=1)
        if j + 1 < block_t:
            _push_tiles(whh_ref)
        pre_i = pltpu.matmul_pop(0, (8, _H), jnp.float32, mxu_index=0)
        pre_g = pltpu.matmul_pop(0, (8, _H), jnp.float32, mxu_index=1)
        pre_f = pltpu.matmul_pop(8, (8, _H), jnp.float32, mxu_index=0)
        pre_o = pltpu.matmul_pop(8, (8, _H), jnp.float32, mxu_index=1)
        i_g = jax.nn.sigmoid(pre_i + g_ref[j:j + 1, 0 * _H:1 * _H])
        f_g = jax.nn.sigmoid(pre_f + g_ref[j:j + 1, 1 * _H:2 * _H])
        g_g = jnp.tanh(pre_g + g_ref[j:j + 1, 2 * _H:3 * _H])
        o_g = jax.nn.sigmoid(pre_o + g_ref[j:j + 1, 3 * _H:4 * _H])
        c = f_g * c + i_g * g_g
        h = o_g * jnp.tanh(c)
        out_ref[j:j + 1, :] = h[0:1, :]
    h_scr[...] = h
    c_scr[...] = c

    @pl.when(blk == pl.num_programs(0) - 1)
    def _():
        c_out_ref[...] = c[0:1, :]


def _pick_block(n, candidates):
    for c in candidates:
        if n % c == 0:
            return c
    return 1


def kernel(emb, w, b, token_ids, h0, c0):
    S = token_ids.shape[0]
    H = _H

    x = emb[token_ids.astype(jnp.int32)]

    wih = pl.pallas_call(
        _prep_kernel,
        grid=(1,),
        in_specs=[pl.BlockSpec((H, _G4), lambda i: (0, 0))],
        out_specs=pl.BlockSpec((H, _G4), lambda i: (0, 0)),
        out_shape=jax.ShapeDtypeStruct((H, _G4), jnp.bfloat16),
    )(w)

    bt1 = _pick_block(S, (256, 128, 64, 32, 16, 8))
    s_pad = S
    if S % bt1:
        s_pad = (S // bt1 + 1) * bt1
        x = jnp.zeros((s_pad, H), x.dtype).at[:S].set(x)
    gmat = pl.pallas_call(
        _proj_kernel,
        grid=(s_pad // bt1,),
        in_specs=[
            pl.BlockSpec((bt1, H), lambda i: (i, 0)),
            pl.BlockSpec((H, _G4), lambda i: (0, 0)),
            pl.BlockSpec((1, _G4), lambda i: (0, 0)),
        ],
        out_specs=pl.BlockSpec((bt1, _G4), lambda i: (i, 0)),
        out_shape=jax.ShapeDtypeStruct((s_pad, _G4), jnp.float32),
        compiler_params=pltpu.CompilerParams(
            dimension_semantics=("parallel",)),
    )(x, wih, b)
    if s_pad != S:
        gmat = gmat[:S]

    bt2 = _pick_block(S, (64, 32, 16, 8, 4, 2))
    h0_2d = h0.reshape(1, H)
    c0_2d = c0.reshape(1, H)
    outputs, c_final = pl.pallas_call(
        functools.partial(_recur_kernel, block_t=bt2),
        grid=(S // bt2,),
        in_specs=[
            pl.BlockSpec((bt2, _G4), lambda i: (i, 0)),
            pl.BlockSpec((H, _G4), lambda i: (1, 0)),
            pl.BlockSpec((1, H), lambda i: (0, 0)),
            pl.BlockSpec((1, H), lambda i: (0, 0)),
        ],
        out_specs=[
            pl.BlockSpec((bt2, H), lambda i: (i, 0)),
            pl.BlockSpec((1, H), lambda i: (0, 0)),
        ],
        out_shape=(jax.ShapeDtypeStruct((S, H), jnp.float32),
                   jax.ShapeDtypeStruct((1, H), jnp.float32)),
        scratch_shapes=[
            pltpu.VMEM((8, H), jnp.float32),
            pltpu.VMEM((8, H), jnp.float32),
        ],
        compiler_params=pltpu.CompilerParams(
            dimension_semantics=("arbitrary",)),
    )(gmat, w, h0_2d, c0_2d)

    outputs = outputs.reshape(S, 1, H)
    h_final = outputs[-1].reshape(1, 1, H)
    c_final = c_final.reshape(1, 1, H)
    return outputs, (h_final, c_final)


# speedup vs baseline: 6.4932x; 1.0475x over previous
import functools

import jax
import jax.numpy as jnp
from jax.experimental import pallas as pl
from jax.experimental.pallas import tpu as pltpu

_H = 256
_G4 = 4 * _H


def _halve_ifo(m):
    return jnp.concatenate(
        [m[:, :2 * _H] * 0.5, m[:, 2 * _H:3 * _H], m[:, 3 * _H:] * 0.5],
        axis=1)


def _prep_kernel(w_ref, b_ref, wih_ref, whh_ref, br_ref):
    wih_ref[...] = _halve_ifo(w_ref[:_H]).astype(jnp.bfloat16)
    whh_ref[...] = _halve_ifo(w_ref[_H:])
    br_ref[...] = _halve_ifo(b_ref[...])


def _proj_kernel(x_ref, wih_ref, b_ref, g_ref):
    g_ref[...] = (
        jnp.dot(x_ref[...].astype(jnp.bfloat16), wih_ref[...],
                preferred_element_type=jnp.float32)
        + b_ref[...]
    )


def _push_tiles(whh_ref):
    pltpu.matmul_push_rhs(whh_ref[:, 0 * _H:1 * _H],
                          staging_register=0, mxu_index=0)
    pltpu.matmul_push_rhs(whh_ref[:, 2 * _H:3 * _H],
                          staging_register=0, mxu_index=1)
    pltpu.matmul_push_rhs(whh_ref[:, 1 * _H:2 * _H],
                          staging_register=1, mxu_index=0)
    pltpu.matmul_push_rhs(whh_ref[:, 3 * _H:4 * _H],
                          staging_register=1, mxu_index=1)


def _recur_kernel(g_ref, whh_ref, h0_ref, c0_ref, out_ref, c_out_ref,
                  h_scr, c_scr, *, block_t):
    blk = pl.program_id(0)

    @pl.when(blk == 0)
    def _():
        h_scr[...] = jnp.broadcast_to(h0_ref[...], (8, _H))
        c_scr[...] = jnp.broadcast_to(c0_ref[...], (8, _H))

    _push_tiles(whh_ref)

    h = h_scr[...]
    c = c_scr[...]
    for j in range(block_t):
        pltpu.matmul_acc_lhs(0, h, mxu_index=0, load_staged_rhs=0)
        pltpu.matmul_acc_lhs(0, h, mxu_index=1, load_staged_rhs=0)
        pltpu.matmul_acc_lhs(8, h, mxu_index=0, load_staged_rhs=1)
        pltpu.matmul_acc_lhs(8, h, mxu_index=1, load_staged_rhs=1)
        if j + 1 < block_t:
            _push_tiles(whh_ref)
        pre_i = pltpu.matmul_pop(0, (8, _H), jnp.float32, mxu_index=0)
        pre_g = pltpu.matmul_pop(0, (8, _H), jnp.float32, mxu_index=1)
        pre_f = pltpu.matmul_pop(8, (8, _H), jnp.float32, mxu_index=0)
        pre_o = pltpu.matmul_pop(8, (8, _H), jnp.float32, mxu_index=1)
        ti = jnp.tanh(pre_i + g_ref[j:j + 1, 0 * _H:1 * _H])
        tf = jnp.tanh(pre_f + g_ref[j:j + 1, 1 * _H:2 * _H])
        g_g = jnp.tanh(pre_g + g_ref[j:j + 1, 2 * _H:3 * _H])
        to = jnp.tanh(pre_o + g_ref[j:j + 1, 3 * _H:4 * _H])
        c = 0.5 * (c * (tf + 1.0) + g_g * (ti + 1.0))
        h = (0.5 * to + 0.5) * jnp.tanh(c)
        out_ref[j:j + 1, :] = h[0:1, :]
    h_scr[...] = h
    c_scr[...] = c

    @pl.when(blk == pl.num_programs(0) - 1)
    def _():
        c_out_ref[...] = c[0:1, :]


def _pick_block(n, candidates):
    for c in candidates:
        if n % c == 0:
            return c
    return 1


def kernel(emb, w, b, token_ids, h0, c0):
    S = token_ids.shape[0]
    H = _H

    x = emb[token_ids.astype(jnp.int32)]

    wih, whh, b_r = pl.pallas_call(
        _prep_kernel,
        grid=(1,),
        in_specs=[pl.BlockSpec((2 * H, _G4), lambda i: (0, 0)),
                  pl.BlockSpec((1, _G4), lambda i: (0, 0))],
        out_specs=[pl.BlockSpec((H, _G4), lambda i: (0, 0)),
                   pl.BlockSpec((H, _G4), lambda i: (0, 0)),
                   pl.BlockSpec((1, _G4), lambda i: (0, 0))],
        out_shape=(jax.ShapeDtypeStruct((H, _G4), jnp.bfloat16),
                   jax.ShapeDtypeStruct((H, _G4), jnp.float32),
                   jax.ShapeDtypeStruct((1, _G4), jnp.float32)),
    )(w, b)

    bt1 = _pick_block(S, (256, 128, 64, 32, 16, 8))
    s_pad = S
    if S % bt1:
        s_pad = (S // bt1 + 1) * bt1
        x = jnp.zeros((s_pad, H), x.dtype).at[:S].set(x)
    gmat = pl.pallas_call(
        _proj_kernel,
        grid=(s_pad // bt1,),
        in_specs=[
            pl.BlockSpec((bt1, H), lambda i: (i, 0)),
            pl.BlockSpec((H, _G4), lambda i: (0, 0)),
            pl.BlockSpec((1, _G4), lambda i: (0, 0)),
        ],
        out_specs=pl.BlockSpec((bt1, _G4), lambda i: (i, 0)),
        out_shape=jax.ShapeDtypeStruct((s_pad, _G4), jnp.float32),
        compiler_params=pltpu.CompilerParams(
            dimension_semantics=("parallel",)),
    )(x, wih, b_r)
    if s_pad != S:
        gmat = gmat[:S]

    bt2 = _pick_block(S, (128, 64, 32, 16, 8, 4, 2))
    h0_2d = h0.reshape(1, H)
    c0_2d = c0.reshape(1, H)
    outputs, c_final = pl.pallas_call(
        functools.partial(_recur_kernel, block_t=bt2),
        grid=(S // bt2,),
        in_specs=[
            pl.BlockSpec((bt2, _G4), lambda i: (i, 0)),
            pl.BlockSpec((H, _G4), lambda i: (0, 0)),
            pl.BlockSpec((1, H), lambda i: (0, 0)),
            pl.BlockSpec((1, H), lambda i: (0, 0)),
        ],
        out_specs=[
            pl.BlockSpec((bt2, H), lambda i: (i, 0)),
            pl.BlockSpec((1, H), lambda i: (0, 0)),
        ],
        out_shape=(jax.ShapeDtypeStruct((S, H), jnp.float32),
                   jax.ShapeDtypeStruct((1, H), jnp.float32)),
        scratch_shapes=[
            pltpu.VMEM((8, H), jnp.float32),
            pltpu.VMEM((8, H), jnp.float32),
        ],
        compiler_params=pltpu.CompilerParams(
            dimension_semantics=("arbitrary",)),
    )(gmat, whh, h0_2d, c0_2d)

    outputs = outputs.reshape(S, 1, H)
    h_final = outputs[-1].reshape(1, 1, H)
    c_final = c_final.reshape(1, 1, H)
    return outputs, (h_final, c_final)


# speedup vs baseline: 6.6379x; 1.0223x over previous
import functools

import jax
import jax.numpy as jnp
from jax.experimental import pallas as pl
from jax.experimental.pallas import tpu as pltpu

_H = 256
_G4 = 4 * _H
_A_R0, _A_R1, _A_P0, _A_P1 = 0, 8, 32, 96


def _push_hh(whh_scr):
    pltpu.matmul_push_rhs(whh_scr[:, 0 * _H:1 * _H],
                          staging_register=0, mxu_index=0)
    pltpu.matmul_push_rhs(whh_scr[:, 2 * _H:3 * _H],
                          staging_register=0, mxu_index=1)
    pltpu.matmul_push_rhs(whh_scr[:, 1 * _H:2 * _H],
                          staging_register=1, mxu_index=0)
    pltpu.matmul_push_rhs(whh_scr[:, 3 * _H:4 * _H],
                          staging_register=1, mxu_index=1)


def _fused_kernel(x_ref, w_ref, b_ref, h0_ref, c0_ref,
                  out_ref, c_out_ref, h_out_ref,
                  h_scr, c_scr, whh_scr, g_scr, *, block_t):
    blk = pl.program_id(0)

    @pl.when(blk == 0)
    def _():
        h_scr[...] = jnp.broadcast_to(h0_ref[...], (8, _H))
        c_scr[...] = jnp.broadcast_to(c0_ref[...], (8, _H))
        whh = w_ref[_H:, :]
        whh_scr[:, :2 * _H] = whh[:, :2 * _H] * 0.5
        whh_scr[:, 2 * _H:3 * _H] = whh[:, 2 * _H:3 * _H]
        whh_scr[:, 3 * _H:] = whh[:, 3 * _H:] * 0.5

    xb = x_ref[...]
    pltpu.matmul_push_rhs(w_ref[:_H, 0 * _H:1 * _H],
                          staging_register=0, mxu_index=0)
    pltpu.matmul_push_rhs(w_ref[:_H, 2 * _H:3 * _H],
                          staging_register=0, mxu_index=1)
    pltpu.matmul_acc_lhs(_A_P0, xb, mxu_index=0, load_staged_rhs=0)
    pltpu.matmul_acc_lhs(_A_P0, xb, mxu_index=1, load_staged_rhs=0)
    pltpu.matmul_push_rhs(w_ref[:_H, 1 * _H:2 * _H],
                          staging_register=0, mxu_index=0)
    pltpu.matmul_push_rhs(w_ref[:_H, 3 * _H:4 * _H],
                          staging_register=0, mxu_index=1)
    pltpu.matmul_acc_lhs(_A_P1, xb, mxu_index=0, load_staged_rhs=0)
    pltpu.matmul_acc_lhs(_A_P1, xb, mxu_index=1, load_staged_rhs=0)
    _push_hh(whh_scr)
    gi = pltpu.matmul_pop(_A_P0, (block_t, _H), jnp.float32, mxu_index=0)
    gg = pltpu.matmul_pop(_A_P0, (block_t, _H), jnp.float32, mxu_index=1)
    gf = pltpu.matmul_pop(_A_P1, (block_t, _H), jnp.float32, mxu_index=0)
    go = pltpu.matmul_pop(_A_P1, (block_t, _H), jnp.float32, mxu_index=1)
    b = b_ref[...]
    g_scr[:, 0 * _H:1 * _H] = (gi + b[:, 0 * _H:1 * _H]) * 0.5
    g_scr[:, 1 * _H:2 * _H] = (gf + b[:, 1 * _H:2 * _H]) * 0.5
    g_scr[:, 2 * _H:3 * _H] = gg + b[:, 2 * _H:3 * _H]
    g_scr[:, 3 * _H:4 * _H] = (go + b[:, 3 * _H:4 * _H]) * 0.5

    h = h_scr[...]
    c = c_scr[...]
    for j in range(block_t):
        pltpu.matmul_acc_lhs(_A_R0, h, mxu_index=0, load_staged_rhs=0)
        pltpu.matmul_acc_lhs(_A_R0, h, mxu_index=1, load_staged_rhs=0)
        pltpu.matmul_acc_lhs(_A_R1, h, mxu_index=0, load_staged_rhs=1)
        pltpu.matmul_acc_lhs(_A_R1, h, mxu_index=1, load_staged_rhs=1)
        if j + 1 < block_t:
            _push_hh(whh_scr)
        pre_i = pltpu.matmul_pop(_A_R0, (8, _H), jnp.float32, mxu_index=0)
        pre_g = pltpu.matmul_pop(_A_R0, (8, _H), jnp.float32, mxu_index=1)
        pre_f = pltpu.matmul_pop(_A_R1, (8, _H), jnp.float32, mxu_index=0)
        pre_o = pltpu.matmul_pop(_A_R1, (8, _H), jnp.float32, mxu_index=1)
        ti = jnp.tanh(pre_i + g_scr[j:j + 1, 0 * _H:1 * _H])
        tf = jnp.tanh(pre_f + g_scr[j:j + 1, 1 * _H:2 * _H])
        g_g = jnp.tanh(pre_g + g_scr[j:j + 1, 2 * _H:3 * _H])
        to = jnp.tanh(pre_o + g_scr[j:j + 1, 3 * _H:4 * _H])
        c = c * (0.5 * tf + 0.5) + g_g * (0.5 * ti + 0.5)
        h = (0.5 * to + 0.5) * jnp.tanh(c)
        out_ref[j:j + 1, :] = h[0:1, :]
    h_scr[...] = h
    c_scr[...] = c

    @pl.when(blk == pl.num_programs(0) - 1)
    def _():
        c_out_ref[...] = c[0:1, :]
        h_out_ref[...] = h[0:1, :]


def _pick_block(n, candidates):
    for c in candidates:
        if n % c == 0:
            return c
    return 1


def kernel(emb, w, b, token_ids, h0, c0):
    S = token_ids.shape[0]
    H = _H

    x = emb[token_ids.astype(jnp.int32)]

    bt = _pick_block(S, (128, 64, 32, 16, 8))
    s_pad = S
    if S % bt:
        s_pad = (S // bt + 1) * bt
        x = jnp.zeros((s_pad, H), x.dtype).at[:S].set(x)

    h0_2d = h0.reshape(1, H)
    c0_2d = c0.reshape(1, H)
    outputs, c_final, h_final = pl.pallas_call(
        functools.partial(_fused_kernel, block_t=bt),
        grid=(s_pad // bt,),
        in_specs=[
            pl.BlockSpec((bt, H), lambda i: (i, 0)),
            pl.BlockSpec((2 * H, _G4), lambda i: (0, 0)),
            pl.BlockSpec((1, _G4), lambda i: (0, 0)),
            pl.BlockSpec((1, H), lambda i: (0, 0)),
            pl.BlockSpec((1, H), lambda i: (0, 0)),
        ],
        out_specs=[
            pl.BlockSpec((bt, H), lambda i: (i, 0)),
            pl.BlockSpec((1, H), lambda i: (0, 0)),
            pl.BlockSpec((1, H), lambda i: (0, 0)),
        ],
        out_shape=(jax.ShapeDtypeStruct((s_pad, H), jnp.float32),
                   jax.ShapeDtypeStruct((1, H), jnp.float32),
                   jax.ShapeDtypeStruct((1, H), jnp.float32)),
        scratch_shapes=[
            pltpu.VMEM((8, H), jnp.float32),
            pltpu.VMEM((8, H), jnp.float32),
            pltpu.VMEM((H, _G4), jnp.float32),
            pltpu.VMEM((bt, _G4), jnp.float32),
        ],
        compiler_params=pltpu.CompilerParams(
            dimension_semantics=("arbitrary",)),
    )(x, w, b, h0_2d, c0_2d)

    if s_pad != S:
        outputs = outputs[:S]
    outputs = outputs.reshape(S, 1, H)
    h_final = h_final.reshape(1, 1, H)
    c_final = c_final.reshape(1, 1, H)
    return outputs, (h_final, c_final)


# speedup vs baseline: 6.8997x; 1.0394x over previous
import functools

import jax
import jax.numpy as jnp
from jax.experimental import pallas as pl
from jax.experimental.pallas import tpu as pltpu

_H = 256
_G4 = 4 * _H
_A_R0, _A_R1, _A_P0, _A_P1 = 0, 8, 32, 96
_LAG = 16


def _push_hh(whh_scr):
    pltpu.matmul_push_rhs(whh_scr[:, 0 * _H:1 * _H],
                          staging_register=0, mxu_index=0)
    pltpu.matmul_push_rhs(whh_scr[:, 2 * _H:3 * _H],
                          staging_register=0, mxu_index=1)
    pltpu.matmul_push_rhs(whh_scr[:, 1 * _H:2 * _H],
                          staging_register=1, mxu_index=0)
    pltpu.matmul_push_rhs(whh_scr[:, 3 * _H:4 * _H],
                          staging_register=1, mxu_index=1)


def _row_copy(tok_ref, emb_ref, x_scr, sem, slot, base, r):
    tok = tok_ref[base + r]
    return pltpu.make_async_copy(
        emb_ref.at[tok], x_scr.at[slot, r], sem)


def _fused_kernel(tok_ref, emb_ref, w_ref, b_ref, h0_ref, c0_ref,
                  out_ref, c_out_ref, h_out_ref,
                  h_scr, c_scr, whh_scr, g_scr, x_scr, sem, *, block_t):
    blk = pl.program_id(0)
    nb = pl.num_programs(0)
    slot = jax.lax.rem(blk, 2)
    nslot = jax.lax.rem(blk + 1, 2)

    @pl.when(blk == 0)
    def _():
        h_scr[...] = jnp.broadcast_to(h0_ref[...], (8, _H))
        c_scr[...] = jnp.broadcast_to(c0_ref[...], (8, _H))
        whh = w_ref[_H:, :]
        whh_scr[:, :2 * _H] = whh[:, :2 * _H] * 0.5
        whh_scr[:, 2 * _H:3 * _H] = whh[:, 2 * _H:3 * _H]
        whh_scr[:, 3 * _H:] = whh[:, 3 * _H:] * 0.5
        for r in range(block_t):
            _row_copy(tok_ref, emb_ref, x_scr, sem, 0, 0, r).start()
        for r in range(block_t):
            _row_copy(tok_ref, emb_ref, x_scr, sem, 0, 0, r).wait()

    xb = x_scr[slot]
    pltpu.matmul_push_rhs(w_ref[:_H, 0 * _H:1 * _H],
                          staging_register=0, mxu_index=0)
    pltpu.matmul_push_rhs(w_ref[:_H, 2 * _H:3 * _H],
                          staging_register=0, mxu_index=1)
    pltpu.matmul_acc_lhs(_A_P0, xb, mxu_index=0, load_staged_rhs=0)
    pltpu.matmul_acc_lhs(_A_P0, xb, mxu_index=1, load_staged_rhs=0)
    pltpu.matmul_push_rhs(w_ref[:_H, 1 * _H:2 * _H],
                          staging_register=0, mxu_index=0)
    pltpu.matmul_push_rhs(w_ref[:_H, 3 * _H:4 * _H],
                          staging_register=0, mxu_index=1)
    pltpu.matmul_acc_lhs(_A_P1, xb, mxu_index=0, load_staged_rhs=0)
    pltpu.matmul_acc_lhs(_A_P1, xb, mxu_index=1, load_staged_rhs=0)
    _push_hh(whh_scr)
    gi = pltpu.matmul_pop(_A_P0, (block_t, _H), jnp.float32, mxu_index=0)
    gg = pltpu.matmul_pop(_A_P0, (block_t, _H), jnp.float32, mxu_index=1)
    gf = pltpu.matmul_pop(_A_P1, (block_t, _H), jnp.float32, mxu_index=0)
    go = pltpu.matmul_pop(_A_P1, (block_t, _H), jnp.float32, mxu_index=1)
    b = b_ref[...]
    g_scr[:, 0 * _H:1 * _H] = (gi + b[:, 0 * _H:1 * _H]) * 0.5
    g_scr[:, 1 * _H:2 * _H] = (gf + b[:, 1 * _H:2 * _H]) * 0.5
    g_scr[:, 2 * _H:3 * _H] = gg + b[:, 2 * _H:3 * _H]
    g_scr[:, 3 * _H:4 * _H] = (go + b[:, 3 * _H:4 * _H]) * 0.5

    nbase = (blk + 1) * block_t
    not_last = blk + 1 < nb

    h = h_scr[...]
    c = c_scr[...]
    for j in range(block_t):
        pltpu.matmul_acc_lhs(_A_R0, h, mxu_index=0, load_staged_rhs=0)
        pltpu.matmul_acc_lhs(_A_R0, h, mxu_index=1, load_staged_rhs=0)
        pltpu.matmul_acc_lhs(_A_R1, h, mxu_index=0, load_staged_rhs=1)
        pltpu.matmul_acc_lhs(_A_R1, h, mxu_index=1, load_staged_rhs=1)
        if j + 1 < block_t:
            _push_hh(whh_scr)

        @pl.when(not_last)
        def _():
            _row_copy(tok_ref, emb_ref, x_scr, sem, nslot, nbase, j).start()
            if j >= _LAG:
                _row_copy(tok_ref, emb_ref, x_scr, sem,
                          nslot, nbase, j - _LAG).wait()

        pre_i = pltpu.matmul_pop(_A_R0, (8, _H), jnp.float32, mxu_index=0)
        pre_g = pltpu.matmul_pop(_A_R0, (8, _H), jnp.float32, mxu_index=1)
        pre_f = pltpu.matmul_pop(_A_R1, (8, _H), jnp.float32, mxu_index=0)
        pre_o = pltpu.matmul_pop(_A_R1, (8, _H), jnp.float32, mxu_index=1)
        ti = jnp.tanh(pre_i + g_scr[j:j + 1, 0 * _H:1 * _H])
        tf = jnp.tanh(pre_f + g_scr[j:j + 1, 1 * _H:2 * _H])
        g_g = jnp.tanh(pre_g + g_scr[j:j + 1, 2 * _H:3 * _H])
        to = jnp.tanh(pre_o + g_scr[j:j + 1, 3 * _H:4 * _H])
        c = c * (0.5 * tf + 0.5) + g_g * (0.5 * ti + 0.5)
        h = (0.5 * to + 0.5) * jnp.tanh(c)
        out_ref[j:j + 1, :] = h[0:1, :]
    h_scr[...] = h
    c_scr[...] = c

    @pl.when(not_last)
    def _():
        for r in range(block_t - _LAG, block_t):
            _row_copy(tok_ref, emb_ref, x_scr, sem, nslot, nbase, r).wait()

    @pl.when(blk == nb - 1)
    def _():
        c_out_ref[...] = c[0:1, :]
        h_out_ref[...] = h[0:1, :]


def _pick_block(n, candidates):
    for c in candidates:
        if n % c == 0:
            return c
    return 1


def kernel(emb, w, b, token_ids, h0, c0):
    S = token_ids.shape[0]
    H = _H

    bt = _pick_block(S, (128, 64, 32))
    tok = token_ids.astype(jnp.int32)
    s_pad = S
    if S % bt:
        s_pad = (S // bt + 1) * bt
        tok = jnp.zeros((s_pad,), jnp.int32).at[:S].set(tok)

    h0_2d = h0.reshape(1, H)
    c0_2d = c0.reshape(1, H)
    grid_spec = pltpu.PrefetchScalarGridSpec(
        num_scalar_prefetch=1,
        grid=(s_pad // bt,),
        in_specs=[
            pl.BlockSpec(memory_space=pltpu.MemorySpace.HBM),
            pl.BlockSpec((2 * H, _G4), lambda i, tok_ref: (0, 0)),
            pl.BlockSpec((1, _G4), lambda i, tok_ref: (0, 0)),
            pl.BlockSpec((1, H), lambda i, tok_ref: (0, 0)),
            pl.BlockSpec((1, H), lambda i, tok_ref: (0, 0)),
        ],
        out_specs=[
            pl.BlockSpec((bt, H), lambda i, tok_ref: (i, 0)),
            pl.BlockSpec((1, H), lambda i, tok_ref: (0, 0)),
            pl.BlockSpec((1, H), lambda i, tok_ref: (0, 0)),
        ],
        scratch_shapes=[
            pltpu.VMEM((8, H), jnp.float32),
            pltpu.VMEM((8, H), jnp.float32),
            pltpu.VMEM((H, _G4), jnp.float32),
            pltpu.VMEM((bt, _G4), jnp.float32),
            pltpu.VMEM((2, bt, H), jnp.float32),
            pltpu.SemaphoreType.DMA,
        ],
    )
    outputs, c_final, h_final = pl.pallas_call(
        functools.partial(_fused_kernel, block_t=bt),
        grid_spec=grid_spec,
        out_shape=(jax.ShapeDtypeStruct((s_pad, H), jnp.float32),
                   jax.ShapeDtypeStruct((1, H), jnp.float32),
                   jax.ShapeDtypeStruct((1, H), jnp.float32)),
        compiler_params=pltpu.CompilerParams(
            dimension_semantics=("arbitrary",)),
    )(tok, emb, w, b, h0_2d, c0_2d)

    if s_pad != S:
        h_final = outputs[S - 1:S]
        outputs = outputs[:S]
    outputs = outputs.reshape(S, 1, H)
    h_final = h_final.reshape(1, 1, H)
    c_final = c_final.reshape(1, 1, H)
    return outputs, (h_final, c_final)


# speedup vs baseline: 6.8999x; 1.0000x over previous
import functools

import jax
import jax.numpy as jnp
from jax.experimental import pallas as pl
from jax.experimental.pallas import tpu as pltpu

_H = 256
_G4 = 4 * _H
_A_R0, _A_R1, _A_P0, _A_P1 = 0, 2, 32, 96
_LAG = 16


def _push_hh(whh_scr):
    pltpu.matmul_push_rhs(whh_scr[:, 0 * _H:1 * _H],
                          staging_register=0, mxu_index=0)
    pltpu.matmul_push_rhs(whh_scr[:, 2 * _H:3 * _H],
                          staging_register=0, mxu_index=1)
    pltpu.matmul_push_rhs(whh_scr[:, 1 * _H:2 * _H],
                          staging_register=1, mxu_index=0)
    pltpu.matmul_push_rhs(whh_scr[:, 3 * _H:4 * _H],
                          staging_register=1, mxu_index=1)


def _row_copy(tok_ref, emb_ref, x_scr, sem, slot, base, r):
    tok = tok_ref[base + r]
    return pltpu.make_async_copy(
        emb_ref.at[tok], x_scr.at[slot, r], sem)


def _fused_kernel(tok_ref, emb_ref, w_ref, b_ref, h0_ref, c0_ref,
                  out_ref, c_out_ref, h_out_ref,
                  h_scr, c_scr, whh_scr, g_scr, x_scr, sem, *, block_t):
    blk = pl.program_id(0)
    nb = pl.num_programs(0)
    slot = jax.lax.rem(blk, 2)
    nslot = jax.lax.rem(blk + 1, 2)

    @pl.when(blk == 0)
    def _():
        h_scr[...] = jnp.broadcast_to(h0_ref[...] * 2.0, (8, _H))
        c_scr[...] = jnp.broadcast_to(c0_ref[...], (8, _H))
        whh = w_ref[_H:, :]
        whh_scr[:, :2 * _H] = whh[:, :2 * _H] * 0.25
        whh_scr[:, 2 * _H:3 * _H] = whh[:, 2 * _H:3 * _H] * 0.5
        whh_scr[:, 3 * _H:] = whh[:, 3 * _H:] * 0.25
        for r in range(block_t):
            _row_copy(tok_ref, emb_ref, x_scr, sem, 0, 0, r).start()
        for r in range(block_t):
            _row_copy(tok_ref, emb_ref, x_scr, sem, 0, 0, r).wait()

    xb = x_scr[slot]
    pltpu.matmul_push_rhs(w_ref[:_H, 0 * _H:1 * _H],
                          staging_register=0, mxu_index=0)
    pltpu.matmul_push_rhs(w_ref[:_H, 2 * _H:3 * _H],
                          staging_register=0, mxu_index=1)
    pltpu.matmul_acc_lhs(_A_P0, xb, mxu_index=0, load_staged_rhs=0)
    pltpu.matmul_acc_lhs(_A_P0, xb, mxu_index=1, load_staged_rhs=0)
    pltpu.matmul_push_rhs(w_ref[:_H, 1 * _H:2 * _H],
                          staging_register=0, mxu_index=0)
    pltpu.matmul_push_rhs(w_ref[:_H, 3 * _H:4 * _H],
                          staging_register=0, mxu_index=1)
    pltpu.matmul_acc_lhs(_A_P1, xb, mxu_index=0, load_staged_rhs=0)
    pltpu.matmul_acc_lhs(_A_P1, xb, mxu_index=1, load_staged_rhs=0)
    _push_hh(whh_scr)
    gi = pltpu.matmul_pop(_A_P0, (block_t, _H), jnp.float32, mxu_index=0)
    gg = pltpu.matmul_pop(_A_P0, (block_t, _H), jnp.float32, mxu_index=1)
    gf = pltpu.matmul_pop(_A_P1, (block_t, _H), jnp.float32, mxu_index=0)
    go = pltpu.matmul_pop(_A_P1, (block_t, _H), jnp.float32, mxu_index=1)
    b = b_ref[...]
    g_scr[:, 0 * _H:1 * _H] = (gi + b[:, 0 * _H:1 * _H]) * 0.5
    g_scr[:, 1 * _H:2 * _H] = (gf + b[:, 1 * _H:2 * _H]) * 0.5
    g_scr[:, 2 * _H:3 * _H] = gg + b[:, 2 * _H:3 * _H]
    g_scr[:, 3 * _H:4 * _H] = (go + b[:, 3 * _H:4 * _H]) * 0.5

    nbase = (blk + 1) * block_t
    not_last = blk + 1 < nb

    h = h_scr[...]
    c = c_scr[...]
    for j in range(block_t):
        pltpu.matmul_acc_lhs(_A_R0, h, mxu_index=0, load_staged_rhs=0)
        pltpu.matmul_acc_lhs(_A_R0, h, mxu_index=1, load_staged_rhs=0)
        pltpu.matmul_acc_lhs(_A_R1, h, mxu_index=0, load_staged_rhs=1)
        pltpu.matmul_acc_lhs(_A_R1, h, mxu_index=1, load_staged_rhs=1)
        if j + 1 < block_t:
            _push_hh(whh_scr)

        @pl.when(not_last)
        def _():
            _row_copy(tok_ref, emb_ref, x_scr, sem, nslot, nbase, j).start()
            if j >= _LAG:
                _row_copy(tok_ref, emb_ref, x_scr, sem,
                          nslot, nbase, j - _LAG).wait()

        v0 = pltpu.matmul_pop(_A_R0, (16, _H), jnp.float32, mxu_index=0)
        v1 = pltpu.matmul_pop(_A_R0, (16, _H), jnp.float32, mxu_index=1)
        ti = jnp.tanh(v0[0:8] + g_scr[j:j + 1, 0 * _H:1 * _H])
        tf = jnp.tanh(v0[8:16] + g_scr[j:j + 1, 1 * _H:2 * _H])
        g_g = jnp.tanh(v1[0:8] + g_scr[j:j + 1, 2 * _H:3 * _H])
        to = jnp.tanh(v1[8:16] + g_scr[j:j + 1, 3 * _H:4 * _H])
        c = c * (0.5 * tf + 0.5) + g_g * (0.5 * ti + 0.5)
        h = (to + 1.0) * jnp.tanh(c)
        out_ref[j:j + 1, :] = 0.5 * h[0:1, :]
    h_scr[...] = h
    c_scr[...] = c

    @pl.when(not_last)
    def _():
        for r in range(block_t - _LAG, block_t):
            _row_copy(tok_ref, emb_ref, x_scr, sem, nslot, nbase, r).wait()

    @pl.when(blk == nb - 1)
    def _():
        c_out_ref[...] = c[0:1, :]
        h_out_ref[...] = 0.5 * h[0:1, :]


def _pick_block(n, candidates):
    for c in candidates:
        if n % c == 0:
            return c
    return 1


def kernel(emb, w, b, token_ids, h0, c0):
    S = token_ids.shape[0]
    H = _H

    bt = _pick_block(S, (128, 64, 32))
    tok = token_ids.astype(jnp.int32)
    s_pad = S
    if S % bt:
        s_pad = (S // bt + 1) * bt
        tok = jnp.zeros((s_pad,), jnp.int32).at[:S].set(tok)

    h0_2d = h0.reshape(1, H)
    c0_2d = c0.reshape(1, H)
    grid_spec = pltpu.PrefetchScalarGridSpec(
        num_scalar_prefetch=1,
        grid=(s_pad // bt,),
        in_specs=[
            pl.BlockSpec(memory_space=pltpu.MemorySpace.HBM),
            pl.BlockSpec((2 * H, _G4), lambda i, tok_ref: (0, 0)),
            pl.BlockSpec((1, _G4), lambda i, tok_ref: (0, 0)),
            pl.BlockSpec((1, H), lambda i, tok_ref: (0, 0)),
            pl.BlockSpec((1, H), lambda i, tok_ref: (0, 0)),
        ],
        out_specs=[
            pl.BlockSpec((bt, H), lambda i, tok_ref: (i, 0)),
            pl.BlockSpec((1, H), lambda i, tok_ref: (0, 0)),
            pl.BlockSpec((1, H), lambda i, tok_ref: (0, 0)),
        ],
        scratch_shapes=[
            pltpu.VMEM((8, H), jnp.float32),
            pltpu.VMEM((8, H), jnp.float32),
            pltpu.VMEM((H, _G4), jnp.float32),
            pltpu.VMEM((bt, _G4), jnp.float32),
            pltpu.VMEM((2, bt, H), jnp.float32),
            pltpu.SemaphoreType.DMA,
        ],
    )
    outputs, c_final, h_final = pl.pallas_call(
        functools.partial(_fused_kernel, block_t=bt),
        grid_spec=grid_spec,
        out_shape=(jax.ShapeDtypeStruct((s_pad, H), jnp.float32),
                   jax.ShapeDtypeStruct((1, H), jnp.float32),
                   jax.ShapeDtypeStruct((1, H), jnp.float32)),
        compiler_params=pltpu.CompilerParams(
            dimension_semantics=("arbitrary",)),
    )(tok, emb, w, b, h0_2d, c0_2d)

    if s_pad != S:
        h_final = outputs[S - 1:S]
        outputs = outputs[:S]
    outputs = outputs.reshape(S, 1, H)
    h_final = h_final.reshape(1, 1, H)
    c_final = c_final.reshape(1, 1, H)
    return outputs, (h_final, c_final)


# speedup vs baseline: 6.9993x; 1.0144x over previous
import functools

import jax
import jax.numpy as jnp
from jax.experimental import pallas as pl
from jax.experimental.pallas import tpu as pltpu

_H = 256
_G4 = 4 * _H
_A_R0, _A_R1, _A_P0, _A_P1 = 0, 2, 32, 96
_LAG = 16


def _push_hh(whh_scr):
    pltpu.matmul_push_rhs(whh_scr[:, 0 * _H:1 * _H],
                          staging_register=0, mxu_index=0)
    pltpu.matmul_push_rhs(whh_scr[:, 2 * _H:3 * _H],
                          staging_register=0, mxu_index=1)
    pltpu.matmul_push_rhs(whh_scr[:, 1 * _H:2 * _H],
                          staging_register=1, mxu_index=0)
    pltpu.matmul_push_rhs(whh_scr[:, 3 * _H:4 * _H],
                          staging_register=1, mxu_index=1)


def _row_copy(tok_ref, emb_ref, x_scr, sem, slot, base, r):
    tok = tok_ref[base + r]
    return pltpu.make_async_copy(
        emb_ref.at[tok], x_scr.at[slot, r], sem)


def _fused_kernel(tok_ref, emb_ref, w_ref, b_ref, h0_ref, c0_ref,
                  out_ref, c_out_ref, h_out_ref,
                  h_scr, c_scr, whh_scr, g_scr, x_scr, sem, *, block_t):
    blk = pl.program_id(0)
    nb = pl.num_programs(0)
    slot = jax.lax.rem(blk, 2)
    nslot = jax.lax.rem(blk + 1, 2)

    @pl.when(blk == 0)
    def _():
        h_scr[...] = jnp.broadcast_to(h0_ref[...] * 2.0, (8, _H))
        c_scr[...] = jnp.broadcast_to(c0_ref[...], (8, _H))
        whh = w_ref[_H:, :]
        whh_scr[:, :2 * _H] = whh[:, :2 * _H] * 0.25
        whh_scr[:, 2 * _H:3 * _H] = whh[:, 2 * _H:3 * _H] * 0.5
        whh_scr[:, 3 * _H:] = whh[:, 3 * _H:] * 0.25
        for r in range(block_t):
            _row_copy(tok_ref, emb_ref, x_scr, sem, 0, 0, r).start()
        for r in range(block_t):
            _row_copy(tok_ref, emb_ref, x_scr, sem, 0, 0, r).wait()

    xb = x_scr[slot]
    pltpu.matmul_push_rhs(w_ref[:_H, 0 * _H:1 * _H],
                          staging_register=0, mxu_index=0)
    pltpu.matmul_push_rhs(w_ref[:_H, 2 * _H:3 * _H],
                          staging_register=0, mxu_index=1)
    pltpu.matmul_acc_lhs(_A_P0, xb, mxu_index=0, load_staged_rhs=0)
    pltpu.matmul_acc_lhs(_A_P0, xb, mxu_index=1, load_staged_rhs=0)
    pltpu.matmul_push_rhs(w_ref[:_H, 1 * _H:2 * _H],
                          staging_register=0, mxu_index=0)
    pltpu.matmul_push_rhs(w_ref[:_H, 3 * _H:4 * _H],
                          staging_register=0, mxu_index=1)
    pltpu.matmul_acc_lhs(_A_P1, xb, mxu_index=0, load_staged_rhs=0)
    pltpu.matmul_acc_lhs(_A_P1, xb, mxu_index=1, load_staged_rhs=0)
    _push_hh(whh_scr)
    gi = pltpu.matmul_pop(_A_P0, (block_t, _H), jnp.float32, mxu_index=0)
    gg = pltpu.matmul_pop(_A_P0, (block_t, _H), jnp.float32, mxu_index=1)
    gf = pltpu.matmul_pop(_A_P1, (block_t, _H), jnp.float32, mxu_index=0)
    go = pltpu.matmul_pop(_A_P1, (block_t, _H), jnp.float32, mxu_index=1)
    b = b_ref[...]
    g_scr[:, 0 * _H:1 * _H] = (gi + b[:, 0 * _H:1 * _H]) * 0.5
    g_scr[:, 1 * _H:2 * _H] = (gf + b[:, 1 * _H:2 * _H]) * 0.5
    g_scr[:, 2 * _H:3 * _H] = gg + b[:, 2 * _H:3 * _H]
    g_scr[:, 3 * _H:4 * _H] = (go + b[:, 3 * _H:4 * _H]) * 0.5

    nbase = (blk + 1) * block_t
    not_last = blk + 1 < nb

    h = h_scr[...]
    c = c_scr[...]
    for j in range(block_t):
        pltpu.matmul_acc_lhs(_A_R0, h, mxu_index=0, load_staged_rhs=0)
        pltpu.matmul_acc_lhs(_A_R0, h, mxu_index=1, load_staged_rhs=0)
        pltpu.matmul_acc_lhs(_A_R1, h, mxu_index=0, load_staged_rhs=1)
        pltpu.matmul_acc_lhs(_A_R1, h, mxu_index=1, load_staged_rhs=1)
        if j + 1 < block_t:
            _push_hh(whh_scr)

        @pl.when(not_last)
        def _():
            _row_copy(tok_ref, emb_ref, x_scr, sem, nslot, nbase, j).start()
            if j >= _LAG:
                _row_copy(tok_ref, emb_ref, x_scr, sem,
                          nslot, nbase, j - _LAG).wait()

        v0 = pltpu.matmul_pop(_A_R0, (16, _H), jnp.float32, mxu_index=0)
        v1 = pltpu.matmul_pop(_A_R0, (16, _H), jnp.float32, mxu_index=1)
        ti = jnp.tanh(v0[0:8] + g_scr[j:j + 1, 0 * _H:1 * _H])
        tf = jnp.tanh(v0[8:16] + g_scr[j:j + 1, 1 * _H:2 * _H])
        g_g = jnp.tanh(v1[0:8] + g_scr[j:j + 1, 2 * _H:3 * _H])
        to = jnp.tanh(v1[8:16] + g_scr[j:j + 1, 3 * _H:4 * _H])
        c = c * (0.5 * tf + 0.5) + g_g * (0.5 * ti + 0.5)
        h = (to + 1.0) * jnp.tanh(c)
        out_ref[j:j + 1, :] = 0.5 * h[0:1, :]
    h_scr[...] = h
    c_scr[...] = c

    @pl.when(not_last)
    def _():
        for r in range(block_t - _LAG, block_t):
            _row_copy(tok_ref, emb_ref, x_scr, sem, nslot, nbase, r).wait()

    @pl.when(blk == nb - 1)
    def _():
        c_out_ref[...] = c[0:1, :]
        h_out_ref[...] = 0.5 * h[0:1, :]


def _pick_block(n, candidates):
    for c in candidates:
        if n % c == 0:
            return c
    return 1


def kernel(emb, w, b, token_ids, h0, c0):
    S = token_ids.shape[0]
    H = _H

    bt = _pick_block(S, (256, 128, 64, 32))
    tok = token_ids.astype(jnp.int32)
    s_pad = S
    if S % bt:
        s_pad = (S // bt + 1) * bt
        tok = jnp.zeros((s_pad,), jnp.int32).at[:S].set(tok)

    h0_2d = h0.reshape(1, H)
    c0_2d = c0.reshape(1, H)
    grid_spec = pltpu.PrefetchScalarGridSpec(
        num_scalar_prefetch=1,
        grid=(s_pad // bt,),
        in_specs=[
            pl.BlockSpec(memory_space=pltpu.MemorySpace.HBM),
            pl.BlockSpec((2 * H, _G4), lambda i, tok_ref: (0, 0)),
            pl.BlockSpec((1, _G4), lambda i, tok_ref: (0, 0)),
            pl.BlockSpec((1, H), lambda i, tok_ref: (0, 0)),
            pl.BlockSpec((1, H), lambda i, tok_ref: (0, 0)),
        ],
        out_specs=[
            pl.BlockSpec((bt, H), lambda i, tok_ref: (i, 0)),
            pl.BlockSpec((1, H), lambda i, tok_ref: (0, 0)),
            pl.BlockSpec((1, H), lambda i, tok_ref: (0, 0)),
        ],
        scratch_shapes=[
            pltpu.VMEM((8, H), jnp.float32),
            pltpu.VMEM((8, H), jnp.float32),
            pltpu.VMEM((H, _G4), jnp.float32),
            pltpu.VMEM((bt, _G4), jnp.float32),
            pltpu.VMEM((2, bt, H), jnp.float32),
            pltpu.SemaphoreType.DMA,
        ],
    )
    outputs, c_final, h_final = pl.pallas_call(
        functools.partial(_fused_kernel, block_t=bt),
        grid_spec=grid_spec,
        out_shape=(jax.ShapeDtypeStruct((s_pad, H), jnp.float32),
                   jax.ShapeDtypeStruct((1, H), jnp.float32),
                   jax.ShapeDtypeStruct((1, H), jnp.float32)),
        compiler_params=pltpu.CompilerParams(
            dimension_semantics=("arbitrary",)),
    )(tok, emb, w, b, h0_2d, c0_2d)

    if s_pad != S:
        h_final = outputs[S - 1:S]
        outputs = outputs[:S]
    outputs = outputs.reshape(S, 1, H)
    h_final = h_final.reshape(1, 1, H)
    c_final = c_final.reshape(1, 1, H)
    return outputs, (h_final, c_final)
